# baseline (speedup 1.0000x reference)
import functools

import jax
import jax.numpy as jnp
from jax.experimental import pallas as pl
from jax.experimental.pallas import tpu as pltpu

N = 4096
K = N // 2
LOGN = 12
ROWS = 128


def _qbit(logd):
    return logd + 3 if logd < 9 else logd - 9


def _step(xp, j, k, d, logd):
    m = 1 << _qbit(logd)
    xr = xp.reshape(N // (2 * m), 2, m, ROWS)
    a = xr[:, 0]
    b = xr[:, 1]
    mx = jnp.maximum(a, b)
    mn = jnp.minimum(a, b)
    dm = ((j & k) == 0).reshape(N // (2 * m), 2, m, ROWS)[:, 0]
    sel_lo = jnp.where(dm, mx, mn)
    sel_hi = jnp.where(dm, mn, mx)
    return jnp.stack([sel_lo, sel_hi], axis=1).reshape(N, ROWS)


def _sort_body(x_ref, o_ref):
    x = x_ref[...]
    parts = [jnp.transpose(x[:, h * 512:(h + 1) * 512]) for h in range(8)]
    xp = jnp.stack(parts, axis=1).reshape(N, ROWS)

    q = jax.lax.broadcasted_iota(jnp.int32, (N, ROWS), 0)
    j = (q % 8) * 512 + (q // 8)

    for logk in range(1, LOGN + 1):
        k = 1 << logk
        for logd in range(logk - 1, -1, -1):
            xp = _step(xp, j, k, 1 << logd, logd)

    xs = xp.reshape(N // 8, 8, ROWS)
    for h in range(4):
        o_ref[:, h * 512:(h + 1) * 512] = jnp.transpose(xs[:, h, :])


@jax.jit
def kernel(x):
    b, t, n = x.shape
    rows = b * t
    flat = x.reshape(rows, n)
    out = pl.pallas_call(
        _sort_body,
        grid=(rows // ROWS,),
        in_specs=[pl.BlockSpec((ROWS, N), lambda i: (i, 0))],
        out_specs=pl.BlockSpec((ROWS, K), lambda i: (i, 0)),
        out_shape=jax.ShapeDtypeStruct((rows, K), jnp.float32),
        compiler_params=pltpu.CompilerParams(
            dimension_semantics=("arbitrary",),
        ),
    )(flat)
    return out.reshape(b, t, K)


# speedup vs baseline: 1.3015x; 1.3015x over previous
import functools

import jax
import jax.numpy as jnp
from jax.experimental import pallas as pl
from jax.experimental.pallas import tpu as pltpu

N = 4096
K = N // 2
LOGN = 12
ROWS = 128


def _row_step(xp, j, k, logd):
    m = 1 << (logd + 3)
    xr = xp.reshape(N // (2 * m), 2, m, ROWS)
    a = xr[:, 0]
    b = xr[:, 1]
    mx = jnp.maximum(a, b)
    mn = jnp.minimum(a, b)
    dm = ((j & k) == 0).reshape(N // (2 * m), 2, m, 1)[:, 0]
    sel_lo = jnp.where(dm, mx, mn)
    sel_hi = jnp.where(dm, mn, mx)
    return jnp.stack([sel_lo, sel_hi], axis=1).reshape(N, ROWS)


def _sub_step(xp, j, q, k, logd):
    dp = 1 << (logd - 9)
    d = 1 << logd
    pu = jnp.roll(xp, dp, axis=0)
    pd = jnp.roll(xp, -dp, axis=0)
    lower = (q & dp) == 0
    partner = jnp.where(lower, pd, pu)
    keep_max = ((j & k) == 0) == lower
    return jnp.where(keep_max, jnp.maximum(xp, partner),
                     jnp.minimum(xp, partner))


def _sort_body(x_ref, o_ref):
    x = x_ref[...]
    parts = [jnp.transpose(x[:, h * 512:(h + 1) * 512]) for h in range(8)]
    xp = jnp.stack(parts, axis=1).reshape(N, ROWS)

    q = jax.lax.broadcasted_iota(jnp.int32, (N, 1), 0)
    j = (q % 8) * 512 + (q // 8)

    for logk in range(1, LOGN + 1):
        k = 1 << logk
        for logd in range(logk - 1, -1, -1):
            if logd >= 9:
                xp = _sub_step(xp, j, q, k, logd)
            else:
                xp = _row_step(xp, j, k, logd)

    xs = xp.reshape(N // 8, 8, ROWS)
    for h in range(4):
        o_ref[:, h * 512:(h + 1) * 512] = jnp.transpose(xs[:, h, :])


@jax.jit
def kernel(x):
    b, t, n = x.shape
    rows = b * t
    flat = x.reshape(rows, n)
    out = pl.pallas_call(
        _sort_body,
        grid=(rows // ROWS,),
        in_specs=[pl.BlockSpec((ROWS, N), lambda i: (i, 0))],
        out_specs=pl.BlockSpec((ROWS, K), lambda i: (i, 0)),
        out_shape=jax.ShapeDtypeStruct((rows, K), jnp.float32),
        compiler_params=pltpu.CompilerParams(
            dimension_semantics=("arbitrary",),
        ),
    )(flat)
    return out.reshape(b, t, K)
